# speedup vs baseline: 1.0035x; 1.0035x over previous
import functools

import jax
import jax.numpy as jnp
from jax.experimental import pallas as pl
from jax.experimental.pallas import tpu as pltpu

N = 100000
E = 6400000
NODE_IN = 128
HID = 128
OUT = 128

_BLK = 2000


def _mlp_body(agg_ref, x_ref, W1a_ref, W1x_ref, b1_ref, g1_ref, bt1_ref,
              W2_ref, b2_ref, g2_ref, bt2_ref, W3_ref, b3_ref, out_ref):
    agg = agg_ref[...]
    x = x_ref[...]
    h = (jax.lax.dot_general(agg, W1a_ref[...], (((1,), (0,)), ((), ())),
                             preferred_element_type=jnp.float32)
         + jax.lax.dot_general(x, W1x_ref[...], (((1,), (0,)), ((), ())),
                               preferred_element_type=jnp.float32)
         + b1_ref[...])
    mu = jnp.mean(h, axis=-1, keepdims=True)
    var = jnp.mean((h - mu) ** 2, axis=-1, keepdims=True)
    h = (h - mu) * jax.lax.rsqrt(var + 1e-5) * g1_ref[...] + bt1_ref[...]
    h = h * jax.nn.sigmoid(h)
    h = jax.lax.dot_general(h, W2_ref[...], (((1,), (0,)), ((), ())),
                            preferred_element_type=jnp.float32) + b2_ref[...]
    mu = jnp.mean(h, axis=-1, keepdims=True)
    var = jnp.mean((h - mu) ** 2, axis=-1, keepdims=True)
    h = (h - mu) * jax.lax.rsqrt(var + 1e-5) * g2_ref[...] + bt2_ref[...]
    h = h * jax.nn.sigmoid(h)
    out_ref[...] = jax.lax.dot_general(
        h, W3_ref[...], (((1,), (0,)), ((), ())),
        preferred_element_type=jnp.float32) + b3_ref[...]


def _mlp(agg16, x, W1a, W1x, b1, g1, bt1, W2, b2, g2, bt2, W3, b3):
    grid = (N // _BLK,)
    full = lambda i: (0, 0)
    return pl.pallas_call(
        _mlp_body,
        grid=grid,
        in_specs=[
            pl.BlockSpec((_BLK, 16), lambda i: (i, 0)),
            pl.BlockSpec((_BLK, NODE_IN), lambda i: (i, 0)),
            pl.BlockSpec((16, HID), full),
            pl.BlockSpec((NODE_IN, HID), full),
            pl.BlockSpec((1, HID), full),
            pl.BlockSpec((1, HID), full),
            pl.BlockSpec((1, HID), full),
            pl.BlockSpec((HID, HID), full),
            pl.BlockSpec((1, HID), full),
            pl.BlockSpec((1, HID), full),
            pl.BlockSpec((1, HID), full),
            pl.BlockSpec((HID, OUT), full),
            pl.BlockSpec((1, OUT), full),
        ],
        out_specs=pl.BlockSpec((_BLK, OUT), lambda i: (i, 0)),
        out_shape=jax.ShapeDtypeStruct((N, OUT), jnp.float32),
        compiler_params=pltpu.CompilerParams(
            dimension_semantics=("arbitrary",),
        ),
    )(agg16, x, W1a, W1x, b1, g1, bt1, W2, b2, g2, bt2, W3, b3)


def kernel(x, edge_index, edge_attr, u, batch,
           W1, b1, g1, bt1, W2, b2, g2, bt2, W3, b3):
    col = edge_index[1]
    s = jax.ops.segment_sum(edge_attr, col, num_segments=N)
    mx = jax.ops.segment_max(edge_attr, col, num_segments=N)
    cnt = jax.ops.segment_sum(jnp.ones((E,), jnp.float32), col, num_segments=N)
    mx = jnp.where(cnt[:, None] > 0, mx, 0.0)
    mean = s / jnp.clip(cnt, 1.0)[:, None]
    agg16 = jnp.concatenate(
        [s, mx, mean, jnp.zeros((N, 4), jnp.float32)], axis=1)
    W1a = jnp.concatenate([W1[:12], jnp.zeros((4, HID), jnp.float32)], axis=0)
    W1x = W1[12:]
    return _mlp(agg16, x, W1a, W1x,
                b1.reshape(1, HID), g1.reshape(1, HID), bt1.reshape(1, HID),
                W2, b2.reshape(1, HID), g2.reshape(1, HID), bt2.reshape(1, HID),
                W3, b3.reshape(1, OUT))


# speedup vs baseline: 1.8964x; 1.8898x over previous
import functools

import jax
import jax.numpy as jnp
from jax import lax
from jax.experimental import pallas as pl
from jax.experimental.pallas import tpu as pltpu
from jax.experimental.pallas import tpu_sc as plsc

N = 100000
E = 6400000
NODE_IN = 128
HID = 128
OUT = 128

_BLK = 2048
_C = 2048
_NROWS = E // 128
_NCHUNK = E // _C
_NW = 32
_SL = 6256


_NIOTA = 782


def _scadd_body(cols2d, ea, iota2d, zr4, zr1, o1, out4, out1,
                acc4, acc1, idxv, ea4, ones, zrows4, zb1, st1):
    c = lax.axis_index("c")
    s = lax.axis_index("s")
    w = c * 16 + s

    pltpu.sync_copy(o1, ones)
    pltpu.sync_copy(zr4, zrows4)
    pltpu.sync_copy(zr1, zb1)

    zbase = w * _NIOTA // _NW
    zend = (w + 1) * _NIOTA // _NW

    def _zrow(r, _):
        pltpu.sync_copy(iota2d.at[pl.ds(r, 1)], idxv.at[pl.ds(0, 1)])
        pltpu.sync_copy(zrows4, acc4.at[idxv.at[0]])
        pltpu.sync_copy(zb1, acc1.at[idxv.at[0]])
        return 0

    lax.fori_loop(zbase, zend, _zrow, 0)
    plsc.subcore_barrier()

    base = w * _NCHUNK // _NW
    end = (w + 1) * _NCHUNK // _NW

    def _chunk(g, _):
        pltpu.sync_copy(cols2d.at[pl.ds(g * (_C // 128), _C // 128)], idxv)
        pltpu.sync_copy(ea.at[pl.ds(g * _C, _C)], ea4)
        for j in range(_C // 128):
            pltpu.sync_copy(ea4.at[pl.ds(j * 128, 128)],
                            acc4.at[idxv.at[j]], add=True)
            pltpu.sync_copy(ones, acc1.at[idxv.at[j]], add=True)
        return 0

    lax.fori_loop(base, end, _chunk, 0)
    plsc.subcore_barrier()

    tail = _SL - 3 * _C

    for k in range(3):

        def _cp(off=k * _C):
            src = s * _SL + off
            pltpu.sync_copy(acc4.at[pl.ds(src, _C)], ea4)
            pltpu.sync_copy(ea4, out4.at[pl.ds(c * N + src, _C)])
            pltpu.sync_copy(acc1.at[pl.ds(src, _C)], st1)
            pltpu.sync_copy(st1, out1.at[pl.ds(c * N + src, _C)])

        _cp()

    @pl.when(s < 15)
    def _():
        src = s * _SL + 3 * _C
        pltpu.sync_copy(acc4.at[pl.ds(src, tail)], ea4.at[pl.ds(0, tail)])
        pltpu.sync_copy(ea4.at[pl.ds(0, tail)],
                        out4.at[pl.ds(c * N + src, tail)])
        pltpu.sync_copy(acc1.at[pl.ds(src, tail)], st1.at[pl.ds(0, tail)])
        pltpu.sync_copy(st1.at[pl.ds(0, tail)],
                        out1.at[pl.ds(c * N + src, tail)])

    @pl.when(s == 15)
    def _():
        src = 15 * _SL + 3 * _C
        pltpu.sync_copy(acc4.at[pl.ds(src, 16)], ea4.at[pl.ds(0, 16)])
        pltpu.sync_copy(ea4.at[pl.ds(0, 16)],
                        out4.at[pl.ds(c * N + src, 16)])
        pltpu.sync_copy(acc1.at[pl.ds(src, 16)], st1.at[pl.ds(0, 16)])
        pltpu.sync_copy(st1.at[pl.ds(0, 16)],
                        out1.at[pl.ds(c * N + src, 16)])


@functools.partial(
    pl.kernel,
    out_type=(jax.ShapeDtypeStruct((2 * N, 4), jnp.float32),
              jax.ShapeDtypeStruct((2 * N,), jnp.float32)),
    mesh=plsc.VectorSubcoreMesh(core_axis_name="c", subcore_axis_name="s"),
    compiler_params=pltpu.CompilerParams(use_tc_tiling_on_sc=False),
    scratch_types=[
        pltpu.VMEM_SHARED((N, 4), jnp.float32),
        pltpu.VMEM_SHARED((N,), jnp.float32),
        pltpu.VMEM((_C // 128, 128), jnp.int32),
        pltpu.VMEM((_C, 4), jnp.float32),
        pltpu.VMEM((128,), jnp.float32),
        pltpu.VMEM((128, 4), jnp.float32),
        pltpu.VMEM((128,), jnp.float32),
        pltpu.VMEM((_C,), jnp.float32),
    ],
)
def _scadd(cols2d, ea, iota2d, zr4, zr1, o1, out4, out1,
           acc4, acc1, idxv, ea4, ones, zrows4, zb1, st1):
    _scadd_body(cols2d, ea, iota2d, zr4, zr1, o1, out4, out1,
                acc4, acc1, idxv, ea4, ones, zrows4, zb1, st1)


def _mlp_body(sum4_ref, cnt_ref, mxT_ref, x_ref, W1a_ref, W1x_ref, b1_ref,
              g1_ref, bt1_ref, W2_ref, b2_ref, g2_ref, bt2_ref, W3_ref,
              b3_ref, out_ref):
    s4 = sum4_ref[...]
    cnt = cnt_ref[...]
    mean4 = s4 / jnp.clip(cnt, 1.0, None)
    mx4 = jnp.where(cnt > 0, mxT_ref[...], 0.0)
    zero4 = jnp.zeros_like(s4)
    aggT = jnp.concatenate([s4, mx4, mean4, zero4], axis=0)
    x = x_ref[...]
    h = (jax.lax.dot_general(aggT, W1a_ref[...], (((0,), (0,)), ((), ())),
                             preferred_element_type=jnp.float32)
         + jax.lax.dot_general(x, W1x_ref[...], (((1,), (0,)), ((), ())),
                               preferred_element_type=jnp.float32)
         + b1_ref[...])
    mu = jnp.mean(h, axis=-1, keepdims=True)
    var = jnp.mean((h - mu) ** 2, axis=-1, keepdims=True)
    h = (h - mu) * jax.lax.rsqrt(var + 1e-5) * g1_ref[...] + bt1_ref[...]
    h = h * jax.nn.sigmoid(h)
    h = jax.lax.dot_general(h, W2_ref[...], (((1,), (0,)), ((), ())),
                            preferred_element_type=jnp.float32) + b2_ref[...]
    mu = jnp.mean(h, axis=-1, keepdims=True)
    var = jnp.mean((h - mu) ** 2, axis=-1, keepdims=True)
    h = (h - mu) * jax.lax.rsqrt(var + 1e-5) * g2_ref[...] + bt2_ref[...]
    h = h * jax.nn.sigmoid(h)
    out_ref[...] = jax.lax.dot_general(
        h, W3_ref[...], (((1,), (0,)), ((), ())),
        preferred_element_type=jnp.float32) + b3_ref[...]


def _mlp(sum4, cnt2, mxT, x, W1a, W1x, b1, g1, bt1, W2, b2, g2, bt2, W3, b3):
    grid = (pl.cdiv(N, _BLK),)
    full = lambda i: (0, 0)
    return pl.pallas_call(
        _mlp_body,
        grid=grid,
        in_specs=[
            pl.BlockSpec((4, _BLK), lambda i: (0, i)),
            pl.BlockSpec((1, _BLK), lambda i: (0, i)),
            pl.BlockSpec((4, _BLK), lambda i: (0, i)),
            pl.BlockSpec((_BLK, NODE_IN), lambda i: (i, 0)),
            pl.BlockSpec((16, HID), full),
            pl.BlockSpec((NODE_IN, HID), full),
            pl.BlockSpec((1, HID), full),
            pl.BlockSpec((1, HID), full),
            pl.BlockSpec((1, HID), full),
            pl.BlockSpec((HID, HID), full),
            pl.BlockSpec((1, HID), full),
            pl.BlockSpec((1, HID), full),
            pl.BlockSpec((1, HID), full),
            pl.BlockSpec((HID, OUT), full),
            pl.BlockSpec((1, OUT), full),
        ],
        out_specs=pl.BlockSpec((_BLK, OUT), lambda i: (i, 0)),
        out_shape=jax.ShapeDtypeStruct((N, OUT), jnp.float32),
        compiler_params=pltpu.CompilerParams(
            dimension_semantics=("arbitrary",),
        ),
    )(sum4, cnt2, mxT, x, W1a, W1x, b1, g1, bt1, W2, b2, g2, bt2, W3, b3)


def kernel(x, edge_index, edge_attr, u, batch,
           W1, b1, g1, bt1, W2, b2, g2, bt2, W3, b3):
    col = edge_index[1]
    cols2d = col.reshape(_NROWS, 128)
    iota2d = jnp.minimum(jnp.arange(_NIOTA * 128, dtype=jnp.int32),
                         N - 1).reshape(_NIOTA, 128)
    zr4 = jnp.zeros((128, 4), jnp.float32)
    zr1 = jnp.zeros((128,), jnp.float32)
    o1 = jnp.ones((128,), jnp.float32)
    out4, out1 = _scadd(cols2d, edge_attr, iota2d, zr4, zr1, o1)
    sum4 = (out4[:N] + out4[N:]).T.reshape(4, N)
    cnt2 = (out1[:N] + out1[N:]).reshape(1, N)

    mx = jax.ops.segment_max(edge_attr, col, num_segments=N)
    mxT = mx.T

    W1a = jnp.concatenate([W1[:12], jnp.zeros((4, HID), jnp.float32)], axis=0)
    W1x = W1[12:]
    return _mlp(sum4, cnt2, mxT, x, W1a, W1x,
                b1.reshape(1, HID), g1.reshape(1, HID), bt1.reshape(1, HID),
                W2, b2.reshape(1, HID), g2.reshape(1, HID), bt2.reshape(1, HID),
                W3, b3.reshape(1, OUT))
